# baseline (speedup 1.0000x reference)
import functools

import jax
import jax.numpy as jnp
from jax import lax
from jax.experimental import pallas as pl
from jax.experimental.pallas import tpu as pltpu
from jax.experimental.pallas import tpu_sc as plsc

N = 10000
D = 128
DH = D // 2
E = 320000
NC, NS = 2, 16
EPT = E // NS
K = 80
NCHUNK = EPT // K
NB = 4
SD = 2
L = 16
HR = 640
ZR = 208

_mesh = plsc.VectorSubcoreMesh(core_axis_name="c", subcore_axis_name="s")


@functools.partial(
    pl.kernel,
    mesh=_mesh,
    compiler_params=pltpu.CompilerParams(use_tc_tiling_on_sc=False,
                                         needs_layout_passes=False),
    out_type=[
        jax.ShapeDtypeStruct((N, D), jnp.float32),
        jax.ShapeDtypeStruct((NC, N, DH), jnp.float32),
    ],
    scratch_types=[
        pltpu.VMEM((NCHUNK, K), jnp.int32),
        pltpu.VMEM((NCHUNK, K), jnp.int32),
        pltpu.VMEM((NB, K, DH), jnp.float32),
        pltpu.VMEM((ZR, DH), jnp.float32),
        pltpu.VMEM((HR, L), jnp.float32),
        pltpu.VMEM((HR // L, L), jnp.float32),
        pltpu.VMEM((HR // 128, 128), jnp.int32),
        pltpu.VMEM_SHARED((N, DH), jnp.float32),
        pltpu.VMEM_SHARED((HR, L), jnp.float32),
        pltpu.SemaphoreType.DMA,
        pltpu.SemaphoreType.DMA,
    ],
)
def _sc_smooth(x_hbm, src_hbm, dst_hbm, idn_hbm,
               y_hbm, xh_hbm, src_v, dst_v, rows_v, agg_v, hist_v, degm_v,
               idn_v, agg_s, deg_s, gsem, ssem):
    c = lax.axis_index("c")
    s = lax.axis_index("s")
    xh = xh_hbm.at[c]
    ones = jnp.full((L,), 1.0, jnp.float32)
    zeros = jnp.zeros((L,), jnp.float32)

    pltpu.sync_copy(src_hbm.at[s], src_v)
    pltpu.sync_copy(dst_hbm.at[s], dst_v)
    pltpu.sync_copy(idn_hbm, idn_v)

    def z_agg(g, carry):
        agg_v[g // (DH // L), pl.ds(lax.rem(g, DH // L) * L, L)] = zeros
        return carry

    def z_hist(g, carry):
        hist_v[g, :] = zeros
        return carry

    lax.fori_loop(0, ZR * DH // L, z_agg, 0)
    lax.fori_loop(0, HR, z_hist, 0)
    for p in range(3):
        pltpu.sync_copy(agg_v, agg_s.at[pl.ds((3 * s + p) * ZR, ZR)])

    @pl.when(s == NS - 1)
    def _():
        pltpu.sync_copy(agg_v.at[pl.ds(0, N - 48 * ZR)],
                        agg_s.at[pl.ds(48 * ZR, N - 48 * ZR)])

    pltpu.sync_copy(hist_v.at[pl.ds(0, HR // NS)],
                    deg_s.at[pl.ds(s * (HR // NS), HR // NS)])

    for p in range(5):
        row0 = s * 625 + p * 125
        pltpu.sync_copy(x_hbm.at[pl.ds(row0, 125), pl.ds(c * DH, DH)],
                        agg_v.at[pl.ds(0, 125)])
        pltpu.sync_copy(agg_v.at[pl.ds(0, 125)], xh.at[pl.ds(row0, 125)])

    plsc.subcore_barrier()

    for b in range(NB - SD):
        pltpu.async_copy(xh.at[src_v.at[b]], rows_v.at[b], gsem)

    def body(i, carry):
        b = lax.rem(i, NB)

        @pl.when(i >= SD)
        def _():
            pltpu.make_async_copy(rows_v.at[0], agg_s.at[dst_v.at[0]],
                                  ssem).wait()

        @pl.when(i + NB - SD < NCHUNK)
        def _():
            nxt = i + NB - SD
            pltpu.async_copy(xh.at[src_v.at[nxt]], rows_v.at[lax.rem(nxt, NB)],
                             gsem)

        for g in range(K // L):
            idx = dst_v[i, pl.ds(g * L, L)]
            plsc.addupdate_scatter(
                hist_v,
                [lax.shift_right_logical(idx, 4), lax.bitwise_and(idx, 15)],
                ones)

        pltpu.make_async_copy(xh.at[src_v.at[i]], rows_v.at[b], gsem).wait()
        pltpu.async_copy(rows_v.at[b], agg_s.at[dst_v.at[i]], ssem, add=True)
        return carry

    lax.fori_loop(0, NCHUNK, body, 0)

    for _ in range(SD):
        pltpu.make_async_copy(rows_v.at[0], agg_s.at[dst_v.at[0]],
                              ssem).wait()
    for j in range(HR // 128):
        pltpu.sync_copy(hist_v.at[pl.ds(j * 128, 128)],
                        deg_s.at[idn_v.at[j]], add=True)

    plsc.subcore_barrier()

    def norm_pass(base, rpp):
        pltpu.sync_copy(agg_s.at[pl.ds(base, rpp)], agg_v.at[pl.ds(0, rpp)])
        pltpu.sync_copy(deg_s.at[pl.ds(base // L, rpp // L)],
                        degm_v.at[pl.ds(0, rpp // L)])

        def gbody(g, carry):
            inv = 1.0 / jnp.maximum(degm_v[g, :], 1.0)
            for l in range(L):
                r = g * L + l
                for q in range(DH // L):
                    agg_v[r, pl.ds(q * L, L)] = (
                        agg_v[r, pl.ds(q * L, L)] * inv[l])
            return carry

        lax.fori_loop(0, rpp // L, gbody, 0)
        pltpu.sync_copy(agg_v.at[pl.ds(0, rpp)],
                        y_hbm.at[pl.ds(base, rpp), pl.ds(c * DH, DH)])

    @pl.when(s < NS - 1)
    def _():
        for p in range(3):
            norm_pass(s * 624 + p * ZR, ZR)

    @pl.when(s == NS - 1)
    def _():
        for p in range(4):
            norm_pass(15 * 624 + p * 160, 160)


def kernel(x, edge_index):
    ei = edge_index.astype(jnp.int32)
    src = ei[0].reshape(NS, NCHUNK, K)
    dst = ei[1].reshape(NS, NCHUNK, K)
    idn = jnp.arange(HR, dtype=jnp.int32).reshape(HR // 128, 128)
    y, _ = _sc_smooth(x, src, dst, idn)
    return y


# speedup vs baseline: 1.0930x; 1.0930x over previous
import functools

import jax
import jax.numpy as jnp
from jax import lax
from jax.experimental import pallas as pl
from jax.experimental.pallas import tpu as pltpu
from jax.experimental.pallas import tpu_sc as plsc

N = 10000
D = 128
DH = D // 2
E = 320000
NC, NS = 2, 16
EPT = E // NS
K = 80
NCHUNK = EPT // K
NB = 6
SD = 1
L = 16
HR = 640
ZR = 52

_mesh = plsc.VectorSubcoreMesh(core_axis_name="c", subcore_axis_name="s")


@functools.partial(
    pl.kernel,
    mesh=_mesh,
    compiler_params=pltpu.CompilerParams(use_tc_tiling_on_sc=False,
                                         needs_layout_passes=False,
                                         skip_device_barrier=True),
    out_type=[
        jax.ShapeDtypeStruct((N, D), jnp.float32),
        jax.ShapeDtypeStruct((NC, N, DH), jnp.float32),
    ],
    scratch_types=[
        pltpu.VMEM((NCHUNK, K), jnp.int32),
        pltpu.VMEM((NCHUNK, K), jnp.int32),
        pltpu.VMEM((NB, K, DH), jnp.float32),
        pltpu.VMEM((ZR, DH), jnp.float32),
        pltpu.VMEM((HR, L), jnp.float32),
        pltpu.VMEM((HR // L, L), jnp.float32),
        pltpu.VMEM((HR // 128, 128), jnp.int32),
        pltpu.VMEM_SHARED((N, DH), jnp.float32),
        pltpu.VMEM_SHARED((HR, L), jnp.float32),
        pltpu.SemaphoreType.DMA,
        pltpu.SemaphoreType.DMA,
        pltpu.SemaphoreType.DMA,
    ],
)
def _sc_smooth(x_hbm, src_hbm, dst_hbm, idn_hbm,
               y_hbm, xh_hbm, src_v, dst_v, rows_v, agg_v, hist_v, degm_v,
               idn_v, agg_s, deg_s, gsem, ssem, zsem):
    c = lax.axis_index("c")
    s = lax.axis_index("s")
    xh = xh_hbm.at[c]
    ones = jnp.full((L,), 1.0, jnp.float32)
    zeros = jnp.zeros((L,), jnp.float32)

    h_src = pltpu.async_copy(src_hbm.at[s], src_v, gsem)
    h_dst = pltpu.async_copy(dst_hbm.at[s], dst_v, ssem)
    h_idn = pltpu.async_copy(idn_hbm, idn_v, zsem)

    def z_agg(g, carry):
        agg_v[g // (DH // L), pl.ds(lax.rem(g, DH // L) * L, L)] = zeros
        return carry

    def z_hist(g, carry):
        hist_v[g, :] = zeros
        return carry

    lax.fori_loop(0, ZR * DH // L, z_agg, 0)
    lax.fori_loop(0, HR, z_hist, 0)
    h_src.wait()
    h_dst.wait()
    h_idn.wait()

    h_z = [pltpu.async_copy(agg_v, agg_s.at[pl.ds((12 * s + p) * ZR, ZR)],
                            zsem) for p in range(12)]
    h_z.append(pltpu.async_copy(hist_v.at[pl.ds(0, HR // NS)],
                                deg_s.at[pl.ds(s * (HR // NS), HR // NS)],
                                zsem))

    @pl.when(s == NS - 1)
    def _():
        pltpu.sync_copy(agg_v.at[pl.ds(0, N - 192 * ZR)],
                        agg_s.at[pl.ds(192 * ZR, N - 192 * ZR)])

    SP = [80] * 7 + [65]
    base = [s * 625 + 80 * q for q in range(8)]
    h_r = pltpu.async_copy(
        x_hbm.at[pl.ds(base[0], SP[0]), pl.ds(c * DH, DH)],
        rows_v.at[0].at[pl.ds(0, SP[0])], gsem)
    h_w = None
    for p in range(8):
        b = p % 2
        h_r.wait()
        if h_w is not None:
            h_w.wait()
        h_w = pltpu.async_copy(rows_v.at[b].at[pl.ds(0, SP[p])],
                               xh.at[pl.ds(base[p], SP[p])], ssem)
        if p + 1 < 8:
            h_r = pltpu.async_copy(
                x_hbm.at[pl.ds(base[p + 1], SP[p + 1]), pl.ds(c * DH, DH)],
                rows_v.at[1 - b].at[pl.ds(0, SP[p + 1])], gsem)
    h_w.wait()
    for h in h_z:
        h.wait()

    plsc.subcore_barrier()

    for b in range(NB - SD):
        pltpu.async_copy(xh.at[src_v.at[b]], rows_v.at[b], gsem)

    def body(i, carry):
        b = lax.rem(i, NB)

        @pl.when(i >= SD)
        def _():
            pltpu.make_async_copy(rows_v.at[0], agg_s.at[dst_v.at[0]],
                                  ssem).wait()

        @pl.when(i + NB - SD < NCHUNK)
        def _():
            nxt = i + NB - SD
            pltpu.async_copy(xh.at[src_v.at[nxt]], rows_v.at[lax.rem(nxt, NB)],
                             gsem)

        for g in range(K // L):
            idx = dst_v[i, pl.ds(g * L, L)]
            plsc.addupdate_scatter(
                hist_v,
                [lax.shift_right_logical(idx, 4), lax.bitwise_and(idx, 15)],
                ones)

        pltpu.make_async_copy(xh.at[src_v.at[i]], rows_v.at[b], gsem).wait()
        pltpu.async_copy(rows_v.at[b], agg_s.at[dst_v.at[i]], ssem, add=True)
        return carry

    lax.fori_loop(0, NCHUNK, body, 0)

    for _ in range(SD):
        pltpu.make_async_copy(rows_v.at[0], agg_s.at[dst_v.at[0]],
                              ssem).wait()
    h_m = [pltpu.async_copy(hist_v.at[pl.ds(j * 128, 128)],
                            deg_s.at[idn_v.at[j]], zsem, add=True)
           for j in range(HR // 128)]
    for h in h_m:
        h.wait()

    plsc.subcore_barrier()

    pltpu.sync_copy(deg_s.at[pl.ds(39 * s, HR // L)], degm_v)
    nbase = 624 * s
    h_r = pltpu.async_copy(agg_s.at[pl.ds(nbase, 80)], rows_v.at[0], gsem)
    h_w = None
    for p in range(8):
        b = p % 2
        h_r.wait()
        if h_w is not None:
            h_w.wait()
        if p + 1 < 8:
            h_r = pltpu.async_copy(agg_s.at[pl.ds(nbase + 80 * (p + 1), 80)],
                                   rows_v.at[1 - b], gsem)

        def gbody(g, carry):
            inv = 1.0 / jnp.maximum(degm_v[p * 5 + g, :], 1.0)
            for l in range(L):
                r = g * L + l
                for q in range(DH // L):
                    rows_v[b, r, pl.ds(q * L, L)] = (
                        rows_v[b, r, pl.ds(q * L, L)] * inv[l])
            return carry

        lax.fori_loop(0, 5, gbody, 0)
        h_w = pltpu.async_copy(rows_v.at[b],
                               y_hbm.at[pl.ds(nbase + 80 * p, 80),
                                        pl.ds(c * DH, DH)], ssem)
    h_w.wait()


def kernel(x, edge_index):
    ei = edge_index.astype(jnp.int32)
    src = ei[0].reshape(NS, NCHUNK, K)
    dst = ei[1].reshape(NS, NCHUNK, K)
    idn = jnp.arange(HR, dtype=jnp.int32).reshape(HR // 128, 128)
    y, _ = _sc_smooth(x, src, dst, idn)
    return y
